# baseline (speedup 1.0000x reference)
import numpy as np

import jax
import jax.numpy as jnp
from jax.experimental import pallas as pl
from jax.experimental.pallas import tpu as pltpu

N = 4096
DA = 64
BLK = 512
NB = N // BLK
WPK = BLK // 32
CH = BLK // 32
LOG2E = 1.4426950408889634


def _mega_body(x_ref, a_hbm, a2_hbm, w0_ref, av0_ref, w1_ref, av1_ref,
               dw0_ref, db0_ref, dw1_ref, db1_ref, ow_ref, ob_ref, o_ref,
               abuf, dma_sem,
               wh0_ref, whb0_ref, dt0_ref, dq0_ref, fb0_ref,
               whl_ref, whbl_ref, dtl_ref, dql_ref, fbl_ref,
               x1_ref, x2_ref, pk1_ref, pk2_ref):
    st = pl.program_id(0)
    b = pl.program_id(1)
    g = st * NB + b

    def copy_for(gi, slot):
        blk = jnp.where(gi < NB, gi, gi - NB) * BLK
        src = jnp.where(gi < NB, 0, 1)
        return blk, src

    def start_fetch(gi, slot):
        blk, src = copy_for(gi, slot)

        @pl.when(gi < NB)
        def _():
            pltpu.make_async_copy(a_hbm.at[pl.ds(blk, BLK), :],
                                  abuf.at[slot], dma_sem).start()

        @pl.when(jnp.logical_and(gi >= NB, gi < 2 * NB))
        def _():
            pltpu.make_async_copy(a2_hbm.at[pl.ds(blk, BLK), :],
                                  abuf.at[slot], dma_sem).start()

    @pl.when(g == 0)
    def _():
        start_fetch(g, g % 2)

    @pl.when(st <= 1)
    def _():
        blk, _src = copy_for(g, g % 2)

        @pl.when(st == 0)
        def _():
            pltpu.make_async_copy(a_hbm.at[pl.ds(blk, BLK), :],
                                  abuf.at[g % 2], dma_sem).wait()

        @pl.when(st == 1)
        def _():
            pltpu.make_async_copy(a2_hbm.at[pl.ds(blk, BLK), :],
                                  abuf.at[g % 2], dma_sem).wait()

        start_fetch(g + 1, (g + 1) % 2)

    def fill_scratch(x, w_ref, av_ref, wh_ref, whb_ref, dt_ref, dq_ref, fb_ref):
        wh = jnp.dot(x, w_ref[...], preferred_element_type=jnp.float32)
        wh_ref[...] = wh
        whb_ref[...] = wh.astype(jnp.bfloat16)
        dt = jax.lax.dot_general(
            av_ref[...][DA:, :], wh, (((0,), (1,)), ((), ())),
            preferred_element_type=jnp.float32) * LOG2E
        dt_ref[...] = dt
        dq_ref[...] = 0.2 * dt
        cm = jnp.sum(wh, axis=0, keepdims=True) * (1.0 / N)
        fb_ref[...] = jnp.where(cm > 0, cm, jnp.exp(cm) - 1.0)

    @pl.when(jnp.logical_and(st == 0, b == 0))
    def _():
        fill_scratch(x_ref[...], w0_ref, av0_ref,
                     wh0_ref, whb0_ref, dt0_ref, dq0_ref, fb0_ref)

    @pl.when(jnp.logical_and(st == 2, b == 0))
    def _():
        fill_scratch(x1_ref[...], w1_ref, av1_ref,
                     whl_ref, whbl_ref, dtl_ref, dql_ref, fbl_ref)

    @pl.when(jnp.logical_and(st == 3, b == 0))
    def _():
        fill_scratch(x2_ref[...], w1_ref, av1_ref,
                     whl_ref, whbl_ref, dtl_ref, dql_ref, fbl_ref)

    def attn_block(wh_ref, whb_ref, dt_ref, dq_ref, fb_ref, av_ref, mask_fn):
        wh_blk = wh_ref[pl.ds(b * BLK, BLK), :]
        s = jnp.dot(wh_blk, av_ref[...][:DA, :],
                    preferred_element_type=jnp.float32) * LOG2E
        dtrow = dt_ref[...]
        dmax = jnp.max(dtrow, axis=1, keepdims=True)
        t = s + dmax
        mt = jnp.maximum(t, 0.2 * t)
        s1 = s - mt
        s2 = 0.2 * s - mt
        u = s1 + dtrow
        v = s2 + dq_ref[...]
        q = jnp.exp2(jnp.maximum(u, v))
        p = mask_fn(q)
        denom = jnp.sum(p, axis=1, keepdims=True)
        acc = jnp.dot(p.astype(jnp.bfloat16), whb_ref[...],
                      preferred_element_type=jnp.float32)
        acc = acc * jnp.where(denom > 0, 1.0 / denom, 0.0)
        acc = jnp.where(acc > 0, acc, jnp.exp(acc) - 1.0)
        return jnp.where(denom > 0, acc, fb_ref[...])

    def pack(src_ref, pk_ref):
        acc = src_ref[0:CH, :].astype(jnp.int32)
        for r in range(1, 32):
            acc = acc + (src_ref[CH * r:CH * (r + 1), :].astype(jnp.int32) << r)
        pk_ref[pl.ds(b * WPK, WPK), :] = acc

    def bits_mask(pk_ref):
        def f(q):
            pkb = pk_ref[pl.ds(b * WPK, WPK), :]
            parts = []
            for r in range(32):
                c = jnp.int32(np.int32(np.uint32(1 << r)))
                bit = (pkb & c) != 0
                parts.append(jnp.where(bit, q[CH * r:CH * (r + 1), :], 0.0))
            return jnp.concatenate(parts, axis=0)
        return f

    @pl.when(st == 0)
    def _():
        ablk = abuf.at[g % 2]
        out = attn_block(wh0_ref, whb0_ref, dt0_ref, dq0_ref, fb0_ref, av0_ref,
                         lambda q: q * ablk[...])
        x1_ref[pl.ds(b * BLK, BLK), :] = out
        pack(ablk, pk1_ref)

    @pl.when(st == 1)
    def _():
        ablk = abuf.at[g % 2]
        out = attn_block(wh0_ref, whb0_ref, dt0_ref, dq0_ref, fb0_ref, av0_ref,
                         lambda q: q * ablk[...])
        x2_ref[pl.ds(b * BLK, BLK), :] = out
        pack(ablk, pk2_ref)

    @pl.when(st == 2)
    def _():
        x1_ref[pl.ds(b * BLK, BLK), :] = attn_block(
            whl_ref, whbl_ref, dtl_ref, dql_ref, fbl_ref, av1_ref,
            bits_mask(pk1_ref))

    @pl.when(st == 3)
    def _():
        x2_ref[pl.ds(b * BLK, BLK), :] = attn_block(
            whl_ref, whbl_ref, dtl_ref, dql_ref, fbl_ref, av1_ref,
            bits_mask(pk2_ref))

    @pl.when(jnp.logical_and(st == 3, b == NB - 1))
    def _():
        xg = jnp.sum(x2_ref[...] - x1_ref[...], axis=0, keepdims=True)
        xg = xg * jnp.float32(1.0 / N)
        h = jnp.dot(xg, dw0_ref[...],
                    preferred_element_type=jnp.float32) + db0_ref[...]
        h = jnp.maximum(h, 0.0)
        h = jnp.dot(h, dw1_ref[...],
                    preferred_element_type=jnp.float32) + db1_ref[...]
        h = jnp.maximum(h, 0.0)
        z = jnp.dot(h, ow_ref[...],
                    preferred_element_type=jnp.float32) + ob_ref[...]
        z = z - jnp.max(z, axis=1, keepdims=True)
        pz = jnp.exp(z)
        o_ref[...] = pz / jnp.sum(pz, axis=1, keepdims=True)


def kernel(X, A, A2, W0, a0, W1, a1, d0_w, d0_b, d1_w, d1_b, out_w, out_b):
    n_feat = X.shape[1]
    n_out = out_w.shape[1]
    const = lambda st, b: (0, 0)
    return pl.pallas_call(
        _mega_body,
        grid=(4, NB),
        in_specs=[
            pl.BlockSpec((N, n_feat), const),
            pl.BlockSpec(memory_space=pl.ANY),
            pl.BlockSpec(memory_space=pl.ANY),
            pl.BlockSpec((n_feat, DA), const),
            pl.BlockSpec((2 * DA, 1), const),
            pl.BlockSpec((DA, DA), const),
            pl.BlockSpec((2 * DA, 1), const),
            pl.BlockSpec((DA, 128), const),
            pl.BlockSpec((1, 128), const),
            pl.BlockSpec((128, 128), const),
            pl.BlockSpec((1, 128), const),
            pl.BlockSpec((128, n_out), const),
            pl.BlockSpec((1, n_out), const),
        ],
        out_specs=pl.BlockSpec((1, n_out), const),
        out_shape=jax.ShapeDtypeStruct((1, n_out), jnp.float32),
        scratch_shapes=[
            pltpu.VMEM((2, BLK, N), jnp.float32),
            pltpu.SemaphoreType.DMA,
            pltpu.VMEM((N, DA), jnp.float32),
            pltpu.VMEM((N, DA), jnp.bfloat16),
            pltpu.VMEM((1, N), jnp.float32),
            pltpu.VMEM((1, N), jnp.float32),
            pltpu.VMEM((1, DA), jnp.float32),
            pltpu.VMEM((N, DA), jnp.float32),
            pltpu.VMEM((N, DA), jnp.bfloat16),
            pltpu.VMEM((1, N), jnp.float32),
            pltpu.VMEM((1, N), jnp.float32),
            pltpu.VMEM((1, DA), jnp.float32),
            pltpu.VMEM((N, DA), jnp.float32),
            pltpu.VMEM((N, DA), jnp.float32),
            pltpu.VMEM((N // 32, N), jnp.int32),
            pltpu.VMEM((N // 32, N), jnp.int32),
        ],
    )(X, A, A2, W0, a0, W1, a1, d0_w, d0_b.reshape(1, -1), d1_w,
      d1_b.reshape(1, -1), out_w, out_b.reshape(1, -1))


# speedup vs baseline: 1.2205x; 1.2205x over previous
import jax
import jax.numpy as jnp
from jax.experimental import pallas as pl
from jax.experimental.pallas import tpu as pltpu

N = 4096
DA = 64
BLK = 512
NB = N // BLK
LOG2E = 1.4426950408889634


def _mega_body(x_ref, a_ref, a2_ref, w0_ref, av0_ref, w1_ref, av1_ref,
               dw0_ref, db0_ref, dw1_ref, db1_ref, ow_ref, ob_ref, o_ref,
               wh0_ref, whb0_ref, dt0_ref, dq0_ref, fb0_ref,
               whl_ref, whbl_ref, dtl_ref, dql_ref, fbl_ref,
               x1_ref, x2_ref):
    st = pl.program_id(0)
    b = pl.program_id(1)

    def fill_scratch(x, w_ref, av_ref, wh_ref, whb_ref, dt_ref, dq_ref, fb_ref):
        wh = jnp.dot(x, w_ref[...], preferred_element_type=jnp.float32)
        wh_ref[...] = wh
        whb_ref[...] = wh.astype(jnp.bfloat16)
        dt = jax.lax.dot_general(
            av_ref[...][DA:, :], wh, (((0,), (1,)), ((), ())),
            preferred_element_type=jnp.float32) * LOG2E
        dt_ref[...] = dt
        dq_ref[...] = 0.2 * dt
        cm = jnp.sum(wh, axis=0, keepdims=True) * (1.0 / N)
        fb_ref[...] = jnp.where(cm > 0, cm, jnp.exp(cm) - 1.0)

    @pl.when(jnp.logical_and(st == 0, b == 0))
    def _():
        fill_scratch(x_ref[...], w0_ref, av0_ref,
                     wh0_ref, whb0_ref, dt0_ref, dq0_ref, fb0_ref)

    @pl.when(jnp.logical_and(st == 2, b == 0))
    def _():
        fill_scratch(x1_ref[...], w1_ref, av1_ref,
                     whl_ref, whbl_ref, dtl_ref, dql_ref, fbl_ref)

    @pl.when(jnp.logical_and(st == 3, b == 0))
    def _():
        fill_scratch(x2_ref[...], w1_ref, av1_ref,
                     whl_ref, whbl_ref, dtl_ref, dql_ref, fbl_ref)

    def attn_block(wh_ref, whb_ref, dt_ref, dq_ref, fb_ref, av_ref, mask_ref):
        wh_blk = wh_ref[pl.ds(b * BLK, BLK), :]
        s = jnp.dot(wh_blk, av_ref[...][:DA, :],
                    preferred_element_type=jnp.float32) * LOG2E
        dtrow = dt_ref[...]
        dmax = jnp.max(dtrow, axis=1, keepdims=True)
        t = s + dmax
        mt = jnp.maximum(t, 0.2 * t)
        s1 = s - mt
        s2 = 0.2 * s - mt
        u = s1 + dtrow
        v = s2 + dq_ref[...]
        p = jnp.exp2(jnp.maximum(u, v)) * mask_ref[...]
        denom = jnp.sum(p, axis=1, keepdims=True)
        acc = jnp.dot(p.astype(jnp.bfloat16), whb_ref[...],
                      preferred_element_type=jnp.float32)
        acc = acc * jnp.where(denom > 0, 1.0 / denom, 0.0)
        acc = jnp.where(acc > 0, acc, jnp.exp(acc) - 1.0)
        return jnp.where(denom > 0, acc, fb_ref[...])

    @pl.when(st == 0)
    def _():
        x1_ref[pl.ds(b * BLK, BLK), :] = attn_block(
            wh0_ref, whb0_ref, dt0_ref, dq0_ref, fb0_ref, av0_ref, a_ref)

    @pl.when(st == 1)
    def _():
        x2_ref[pl.ds(b * BLK, BLK), :] = attn_block(
            wh0_ref, whb0_ref, dt0_ref, dq0_ref, fb0_ref, av0_ref, a2_ref)

    @pl.when(st == 2)
    def _():
        x1_ref[pl.ds(b * BLK, BLK), :] = attn_block(
            whl_ref, whbl_ref, dtl_ref, dql_ref, fbl_ref, av1_ref, a_ref)

    @pl.when(st == 3)
    def _():
        x2_ref[pl.ds(b * BLK, BLK), :] = attn_block(
            whl_ref, whbl_ref, dtl_ref, dql_ref, fbl_ref, av1_ref, a2_ref)

    @pl.when(jnp.logical_and(st == 3, b == NB - 1))
    def _():
        xg = jnp.sum(x2_ref[...] - x1_ref[...], axis=0, keepdims=True)
        xg = xg * jnp.float32(1.0 / N)
        h = jnp.dot(xg, dw0_ref[...],
                    preferred_element_type=jnp.float32) + db0_ref[...]
        h = jnp.maximum(h, 0.0)
        h = jnp.dot(h, dw1_ref[...],
                    preferred_element_type=jnp.float32) + db1_ref[...]
        h = jnp.maximum(h, 0.0)
        z = jnp.dot(h, ow_ref[...],
                    preferred_element_type=jnp.float32) + ob_ref[...]
        z = z - jnp.max(z, axis=1, keepdims=True)
        pz = jnp.exp(z)
        o_ref[...] = pz / jnp.sum(pz, axis=1, keepdims=True)


def kernel(X, A, A2, W0, a0, W1, a1, d0_w, d0_b, d1_w, d1_b, out_w, out_b):
    n_feat = X.shape[1]
    n_out = out_w.shape[1]
    const = lambda st, b: (0, 0)
    return pl.pallas_call(
        _mega_body,
        grid=(4, NB),
        in_specs=[
            pl.BlockSpec((N, n_feat), const),
            pl.BlockSpec((BLK, N),
                         lambda st, b: (jnp.where((st == 0) | (st == 2),
                                                  b, NB - 1), 0)),
            pl.BlockSpec((BLK, N),
                         lambda st, b: (jnp.where((st == 1) | (st == 3), b,
                                                  jnp.where(st == 0, 0,
                                                            NB - 1)), 0)),
            pl.BlockSpec((n_feat, DA), const),
            pl.BlockSpec((2 * DA, 1), const),
            pl.BlockSpec((DA, DA), const),
            pl.BlockSpec((2 * DA, 1), const),
            pl.BlockSpec((DA, 128), const),
            pl.BlockSpec((1, 128), const),
            pl.BlockSpec((128, 128), const),
            pl.BlockSpec((1, 128), const),
            pl.BlockSpec((128, n_out), const),
            pl.BlockSpec((1, n_out), const),
        ],
        out_specs=pl.BlockSpec((1, n_out), const),
        out_shape=jax.ShapeDtypeStruct((1, n_out), jnp.float32),
        scratch_shapes=[
            pltpu.VMEM((N, DA), jnp.float32),
            pltpu.VMEM((N, DA), jnp.bfloat16),
            pltpu.VMEM((1, N), jnp.float32),
            pltpu.VMEM((1, N), jnp.float32),
            pltpu.VMEM((1, DA), jnp.float32),
            pltpu.VMEM((N, DA), jnp.float32),
            pltpu.VMEM((N, DA), jnp.bfloat16),
            pltpu.VMEM((1, N), jnp.float32),
            pltpu.VMEM((1, N), jnp.float32),
            pltpu.VMEM((1, DA), jnp.float32),
            pltpu.VMEM((N, DA), jnp.float32),
            pltpu.VMEM((N, DA), jnp.float32),
        ],
    )(X, A, A2, W0, a0, W1, a1, d0_w, d0_b.reshape(1, -1), d1_w,
      d1_b.reshape(1, -1), out_w, out_b.reshape(1, -1))
